# baseline (speedup 1.0000x reference)
import functools

import jax
import jax.numpy as jnp
from jax import lax
from jax.experimental import pallas as pl
from jax.experimental.pallas import tpu as pltpu
from jax.experimental.pallas import tpu_sc as plsc

_L = 16

_GDN = lax.GatherDimensionNumbers(offset_dims=(), collapsed_slice_dims=(0,),
                                  start_index_map=(0,))


def _lane_sum(v, iota):
    for sh in (8, 4, 2, 1):
        g = lax.gather(v, (iota ^ sh)[:, None], _GDN, (1,),
                       mode=lax.GatherScatterMode.PROMISE_IN_BOUNDS)
        v = v + g
    return v


def _sc_body(mf, df, idx, fout, part,
             f0b, f1b, f2b, f3b, dfb, mb, idxb,
             svar, sae, sse, scnt, cbv, cba, cbs, cbc,
             scrv, scra, scrs, sidx,
             sem0, sem1, sem2, sem3, sem4, sem5,
             *, at, k, nch, nc, b, n3):
    wid = lax.axis_index("s") * nc + lax.axis_index("c")
    ck = 3 * k
    nvec = ck // _L
    iota = lax.iota(jnp.int32, _L)
    sel0 = jnp.where(iota == 0, 1.0, 0.0)
    one = jnp.ones((_L,), jnp.float32)
    z = jnp.zeros((_L,), jnp.float32)

    def zero_body(i, _):
        sl = pl.ds(i * _L, _L)
        svar[sl] = z
        sae[sl] = z
        sse[sl] = z
        scnt[sl] = z
        return 0
    lax.fori_loop(0, b, zero_body, 0)
    def zero_cb(i, _):
        sl = pl.ds(i * _L, _L)
        cbv[sl] = z
        cba[sl] = z
        cbs[sl] = z
        cbc[sl] = z
        return 0
    lax.fori_loop(0, (b + _L) // _L, zero_cb, 0)
    for r in (scrv, scra, scrs):
        r[pl.ds(0, _L)] = z
        r[pl.ds(_L, _L)] = z

    def rmw(ref, pos, vec):
        cur = ref[pl.ds(pos, _L)]
        ref[pl.ds(pos, _L)] = cur + vec

    def flush(cur, acn, av, aa, aq):
        base = cur * _L
        rmw(svar, base, av)
        rmw(sae, base, aa)
        rmw(sse, base, aq)
        rmw(scnt, base, acn)

    def chunk_body(c, _):
        ab = wid * at + c * k
        cb = 3 * ab
        c0 = pltpu.async_copy(mf.at[pl.ds(cb, ck)], f0b, sem0)
        c1 = pltpu.async_copy(mf.at[pl.ds(n3 + cb, ck)], f1b, sem1)
        c2 = pltpu.async_copy(mf.at[pl.ds(2 * n3 + cb, ck)], f2b, sem2)
        c3 = pltpu.async_copy(mf.at[pl.ds(3 * n3 + cb, ck)], f3b, sem3)
        c4 = pltpu.async_copy(df.at[pl.ds(cb, ck)], dfb, sem4)
        c5 = pltpu.async_copy(idx.at[pl.ds(ab, k)], idxb.at[pl.ds(0, k)], sem5)
        c0.wait(); c1.wait(); c2.wait(); c3.wait(); c4.wait(); c5.wait()

        def vec_body(j, carry):
            cur, acn, av, aa, aq = carry
            p0 = j * _L
            sl = pl.ds(p0, _L)
            f0 = f0b[sl]
            f1 = f1b[sl]
            f2 = f2b[sl]
            f3 = f3b[sl]
            dv = dfb[sl]
            m = (f0 + f1 + f2 + f3) * 0.25
            mb[sl] = m
            d0 = f0 - m
            d1 = f1 - m
            d2 = f2 - m
            d3 = f3 - m
            var_c = (d0 * d0 + d1 * d1 + d2 * d2 + d3 * d3) * (1.0 / 3.0)
            fd = m - dv
            ae_c = jnp.abs(fd)
            se_c = fd * fd
            s0 = idxb[pl.ds(lax.div(p0, 3), _L)][0]
            s15 = idxb[pl.ds(lax.div(p0 + _L - 1, 3), _L)][0]
            same = jnp.logical_and(s0 == cur, s15 == cur)

            @pl.when(jnp.logical_not(same))
            def _():
                flush(cur, acn, av, aa, aq)
                scrv[pl.ds(0, _L)] = var_c
                scra[pl.ds(0, _L)] = ae_c
                scrs[pl.ds(0, _L)] = se_c
                def lane_body(l, _):
                    s = idxb[pl.ds(lax.div(p0 + l, 3), _L)][0]
                    base = s * _L
                    rmw(svar, base, scrv[pl.ds(l, _L)] * sel0)
                    rmw(sae, base, scra[pl.ds(l, _L)] * sel0)
                    rmw(sse, base, scrs[pl.ds(l, _L)] * sel0)
                    rmw(scnt, base, sel0)
                    return 0
                lax.fori_loop(0, _L, lane_body, 0)

            return (jnp.where(same, cur, s15),
                    jnp.where(same, acn + one, z),
                    jnp.where(same, av + var_c, z),
                    jnp.where(same, aa + ae_c, z),
                    jnp.where(same, aq + se_c, z))

        init = (idxb[pl.ds(0, _L)][0], z, z, z, z)
        cur, acn, av, aa, aq = lax.fori_loop(0, nvec, vec_body, init)
        flush(cur, acn, av, aa, aq)
        pltpu.sync_copy(mb, fout.at[pl.ds(cb, ck)])
        return 0

    lax.fori_loop(0, nch, chunk_body, 0)

    pltpu.sync_copy(idx.at[pl.ds(wid * at, _L)], sidx)
    s_lo = sidx[pl.ds(0, _L)][0]
    s_hi = idxb[pl.ds(k - _L, _L)][_L - 1]

    def seg_body(s, _):
        base = s * _L
        for wide, comp in ((svar, cbv), (sae, cba), (sse, cbs), (scnt, cbc)):
            v = wide[pl.ds(base, _L)]
            rmw(comp, s, _lane_sum(v, iota) * sel0)
        return 0
    lax.fori_loop(s_lo, s_hi + 1, seg_body, 0)

    pb = wid * 4 * b
    pltpu.sync_copy(cbv.at[pl.ds(0, b)], part.at[pl.ds(pb, b)])
    pltpu.sync_copy(cba.at[pl.ds(0, b)], part.at[pl.ds(pb + b, b)])
    pltpu.sync_copy(cbs.at[pl.ds(0, b)], part.at[pl.ds(pb + 2 * b, b)])
    pltpu.sync_copy(cbc.at[pl.ds(0, b)], part.at[pl.ds(pb + 3 * b, b)])


def _sc_forces(mf, df, idx, b, nc, ns, interpret=False):
    n3 = mf.shape[0] // 4
    n = n3 // 3
    w = nc * ns
    assert n % w == 0
    at = n // w
    k = 2000 if at % 2000 == 0 else at
    assert at % k == 0 and k % _L == 0
    nch = at // k
    ck = 3 * k
    mesh = plsc.VectorSubcoreMesh(core_axis_name="c", subcore_axis_name="s",
                                  num_cores=nc, num_subcores=ns)
    f = pl.kernel(
        functools.partial(_sc_body, at=at, k=k, nch=nch, nc=nc, b=b, n3=n3),
        out_type=(jax.ShapeDtypeStruct((n3,), jnp.float32),
                  jax.ShapeDtypeStruct((w * 4 * b,), jnp.float32)),
        mesh=mesh,
        scratch_types=[pltpu.VMEM((ck,), jnp.float32)] * 6
                      + [pltpu.VMEM((k + _L,), jnp.int32)]
                      + [pltpu.VMEM((b * _L,), jnp.float32)] * 4
                      + [pltpu.VMEM((b + _L,), jnp.float32)] * 4
                      + [pltpu.VMEM((2 * _L,), jnp.float32)] * 3
                      + [pltpu.VMEM((_L,), jnp.int32)]
                      + [pltpu.SemaphoreType.DMA] * 6,
        interpret=interpret,
    )
    return f(mf, df, idx)


def _tc_finalize_body(me_ref, de_ref, pp_ref,
                      en_ref, emax_ref, emin_ref, evar_ref, esd_ref,
                      fvar_ref, fsd_ref, eae_ref, ese_ref, fae_ref, fse_ref):
    me = me_ref[...]
    de = de_ref[...]
    p = pp_ref[...]
    s = jnp.sum(p, axis=0)
    cnt = jnp.maximum(s[3:4], 1.0)
    fvar = s[0:1] / cnt
    fvar_ref[...] = fvar
    fsd_ref[...] = jnp.sqrt(fvar)
    fae_ref[...] = s[1:2] / cnt
    fse_ref[...] = s[2:3] / cnt
    m = me.shape[0]
    en = jnp.mean(me, axis=0, keepdims=True)
    en_ref[...] = en
    dev = me - en
    evar = jnp.sum(dev * dev, axis=0, keepdims=True) * (1.0 / (m - 1))
    evar_ref[...] = evar
    esd_ref[...] = jnp.sqrt(evar)
    emax_ref[...] = jnp.broadcast_to(jnp.max(me), (1, 1))
    emin_ref[...] = jnp.broadcast_to(jnp.min(me), (1, 1))
    ed = en - de
    eae_ref[...] = jnp.abs(ed)
    ese_ref[...] = ed * ed


def _tc_finalize(me, de2, part, interpret=False):
    b = me.shape[1]
    vb = jax.ShapeDtypeStruct((1, b), jnp.float32)
    sb = jax.ShapeDtypeStruct((1, 1), jnp.float32)
    return pl.pallas_call(
        _tc_finalize_body,
        out_shape=(vb, sb, sb, vb, vb, vb, vb, vb, vb, vb, vb),
        interpret=interpret,
    )(me, de2, part)


def kernel(model_energies, model_forces, data_energy, data_forces, image_idx):
    m, n, _ = model_forces.shape
    b = model_energies.shape[1]
    info = plsc.get_sparse_core_info()
    nc, ns = info.num_cores, info.num_subcores
    mf = model_forces.reshape(m * n * 3)
    df = data_forces.reshape(n * 3)
    forces_flat, part = _sc_forces(mf, df, image_idx, b, nc, ns)
    (en, emax, emin, evar, esd, fvar, fsd, eae, ese, fae, fse) = _tc_finalize(
        model_energies, data_energy.reshape(1, b),
        part.reshape(nc * ns, 4, b))
    return (en.reshape(b), forces_flat.reshape(n, 3), emax.reshape(1),
            emin.reshape(1), evar.reshape(b), esd.reshape(b), fvar.reshape(b),
            fsd.reshape(b), eae.reshape(b), ese.reshape(b), fae.reshape(b),
            fse.reshape(b))


# speedup vs baseline: 38.8532x; 38.8532x over previous
import functools

import jax
import jax.numpy as jnp
from jax import lax
from jax.experimental import pallas as pl
from jax.experimental.pallas import tpu as pltpu
from jax.experimental.pallas import tpu_sc as plsc

_L = 16

_GDN = lax.GatherDimensionNumbers(offset_dims=(), collapsed_slice_dims=(0,),
                                  start_index_map=(0,))


def _lane_sum(v, iota):
    for sh in (8, 4, 2, 1):
        g = lax.gather(v, (iota ^ sh)[:, None], _GDN, (1,),
                       mode=lax.GatherScatterMode.PROMISE_IN_BOUNDS)
        v = v + g
    return v


def _sc_body(*refs, at, k, nch, nc, b):
    (m00, m01, m02, m10, m11, m12, m20, m21, m22, m30, m31, m32,
     d0, d1, d2, idx,
     f0, f1, f2, part,
     b00, b01, b02, b10, b11, b12, b20, b21, b22, b30, b31, b32,
     db0, db1, db2, ob0, ob1, ob2, idxb,
     svar, sae, sse, scnt, cbv, cba, cbs, cbc,
     scrv, scra, scrs, sidx, sem) = refs
    wid = lax.axis_index("s") * nc + lax.axis_index("c")
    nvec = k // _L
    iota = lax.iota(jnp.int32, _L)
    sel0 = jnp.where(iota == 0, 1.0, 0.0)
    one = jnp.ones((_L,), jnp.float32)
    z = jnp.zeros((_L,), jnp.float32)

    def zero_body(i, _):
        sl = pl.ds(i * _L, _L)
        svar[sl] = z
        sae[sl] = z
        sse[sl] = z
        scnt[sl] = z
        return 0
    lax.fori_loop(0, b, zero_body, 0)

    def zero_cb(i, _):
        sl = pl.ds(i * _L, _L)
        cbv[sl] = z
        cba[sl] = z
        cbs[sl] = z
        cbc[sl] = z
        return 0
    lax.fori_loop(0, (b + _L) // _L, zero_cb, 0)
    for r in (scrv, scra, scrs):
        r[pl.ds(0, _L)] = z
        r[pl.ds(_L, _L)] = z

    def rmw(ref, pos, vec):
        cur = ref[pl.ds(pos, _L)]
        ref[pl.ds(pos, _L)] = cur + vec

    def flush(cur, acn, av, aa, aq):
        base = cur * _L
        rmw(svar, base, av)
        rmw(sae, base, aa)
        rmw(sse, base, aq)
        rmw(scnt, base, acn)

    ins = (m00, m01, m02, m10, m11, m12, m20, m21, m22, m30, m31, m32,
           d0, d1, d2)
    bufs = (b00, b01, b02, b10, b11, b12, b20, b21, b22, b30, b31, b32,
            db0, db1, db2)

    def chunk_body(c, _):
        ab = wid * at + c * k
        cps = [pltpu.async_copy(src.at[pl.ds(ab, k)], dst, sem)
               for src, dst in zip(ins, bufs)]
        cps.append(pltpu.async_copy(idx.at[pl.ds(ab, k)],
                                    idxb.at[pl.ds(0, k)], sem))
        for cp in cps:
            cp.wait()

        def vec_body(j, carry):
            cur, acn, av, aa, aq = carry
            p0 = j * _L
            sl = pl.ds(p0, _L)
            vd0 = db0[sl]
            vd1 = db1[sl]
            vd2 = db2[sl]
            ssd = z
            sae_v = z
            sse_v = z
            for mb, dv, ob in (((b00, b10, b20, b30), vd0, ob0),
                               ((b01, b11, b21, b31), vd1, ob1),
                               ((b02, b12, b22, b32), vd2, ob2)):
                v0 = mb[0][sl]
                v1 = mb[1][sl]
                v2 = mb[2][sl]
                v3 = mb[3][sl]
                mn = (v0 + v1 + v2 + v3) * 0.25
                ob[sl] = mn
                e0 = v0 - mn
                e1 = v1 - mn
                e2 = v2 - mn
                e3 = v3 - mn
                ssd = ssd + (e0 * e0 + e1 * e1 + e2 * e2 + e3 * e3)
                fd = mn - dv
                sae_v = sae_v + jnp.abs(fd)
                sse_v = sse_v + fd * fd
            s0 = idxb[pl.ds(p0, _L)][0]
            s15 = idxb[pl.ds(p0 + _L - 1, _L)][0]
            same = jnp.logical_and(s0 == cur, s15 == cur)

            @pl.when(jnp.logical_not(same))
            def _():
                flush(cur, acn, av, aa, aq)
                scrv[pl.ds(0, _L)] = ssd
                scra[pl.ds(0, _L)] = sae_v
                scrs[pl.ds(0, _L)] = sse_v
                def lane_body(l, _):
                    s = idxb[pl.ds(p0 + l, _L)][0]
                    base = s * _L
                    rmw(svar, base, scrv[pl.ds(l, _L)] * sel0)
                    rmw(sae, base, scra[pl.ds(l, _L)] * sel0)
                    rmw(sse, base, scrs[pl.ds(l, _L)] * sel0)
                    rmw(scnt, base, sel0)
                    return 0
                lax.fori_loop(0, _L, lane_body, 0)

            return (jnp.where(same, cur, s15),
                    jnp.where(same, acn + one, z),
                    jnp.where(same, av + ssd, z),
                    jnp.where(same, aa + sae_v, z),
                    jnp.where(same, aq + sse_v, z))

        init = (idxb[pl.ds(0, _L)][0], z, z, z, z)
        cur, acn, av, aa, aq = lax.fori_loop(0, nvec, vec_body, init)
        flush(cur, acn, av, aa, aq)
        pltpu.sync_copy(ob0, f0.at[pl.ds(ab, k)])
        pltpu.sync_copy(ob1, f1.at[pl.ds(ab, k)])
        pltpu.sync_copy(ob2, f2.at[pl.ds(ab, k)])
        return 0

    lax.fori_loop(0, nch, chunk_body, 0)

    pltpu.sync_copy(idx.at[pl.ds(wid * at, _L)], sidx)
    s_lo = sidx[pl.ds(0, _L)][0]
    s_hi = idxb[pl.ds(k - _L, _L)][_L - 1]

    def seg_body(s, _):
        base = s * _L
        for wide, comp in ((svar, cbv), (sae, cba), (sse, cbs), (scnt, cbc)):
            v = wide[pl.ds(base, _L)]
            rmw(comp, s, _lane_sum(v, iota) * sel0)
        return 0
    lax.fori_loop(s_lo, s_hi + 1, seg_body, 0)

    pb = wid * 4 * b
    pltpu.sync_copy(cbv.at[pl.ds(0, b)], part.at[pl.ds(pb, b)])
    pltpu.sync_copy(cba.at[pl.ds(0, b)], part.at[pl.ds(pb + b, b)])
    pltpu.sync_copy(cbs.at[pl.ds(0, b)], part.at[pl.ds(pb + 2 * b, b)])
    pltpu.sync_copy(cbc.at[pl.ds(0, b)], part.at[pl.ds(pb + 3 * b, b)])


def _sc_forces(planes, dplanes, idx, b, nc, ns, interpret=False):
    n = planes[0].shape[0]
    w = nc * ns
    assert n % w == 0
    at = n // w
    k = 2000 if at % 2000 == 0 else at
    assert at % k == 0 and k % _L == 0
    nch = at // k
    mesh = plsc.VectorSubcoreMesh(core_axis_name="c", subcore_axis_name="s",
                                  num_cores=nc, num_subcores=ns)
    vec = jax.ShapeDtypeStruct((n,), jnp.float32)
    f = pl.kernel(
        functools.partial(_sc_body, at=at, k=k, nch=nch, nc=nc, b=b),
        out_type=(vec, vec, vec,
                  jax.ShapeDtypeStruct((w * 4 * b,), jnp.float32)),
        mesh=mesh,
        scratch_types=[pltpu.VMEM((k,), jnp.float32)] * 18
                      + [pltpu.VMEM((k + _L,), jnp.int32)]
                      + [pltpu.VMEM((b * _L,), jnp.float32)] * 4
                      + [pltpu.VMEM((b + _L,), jnp.float32)] * 4
                      + [pltpu.VMEM((2 * _L,), jnp.float32)] * 3
                      + [pltpu.VMEM((_L,), jnp.int32)]
                      + [pltpu.SemaphoreType.DMA],
        interpret=interpret,
    )
    return f(*planes, *dplanes, idx)


def _tc_finalize_body(me_ref, de_ref, pp_ref,
                      en_ref, emax_ref, emin_ref, evar_ref, esd_ref,
                      fvar_ref, fsd_ref, eae_ref, ese_ref, fae_ref, fse_ref):
    me = me_ref[...]
    de = de_ref[...]
    p = pp_ref[...]
    s = jnp.sum(p, axis=0)
    cnt = jnp.maximum(s[3:4], 1.0)
    fvar = s[0:1] / (9.0 * cnt)
    fvar_ref[...] = fvar
    fsd_ref[...] = jnp.sqrt(fvar)
    fae_ref[...] = s[1:2] / (3.0 * cnt)
    fse_ref[...] = s[2:3] / (3.0 * cnt)
    m = me.shape[0]
    en = jnp.mean(me, axis=0, keepdims=True)
    en_ref[...] = en
    dev = me - en
    evar = jnp.sum(dev * dev, axis=0, keepdims=True) * (1.0 / (m - 1))
    evar_ref[...] = evar
    esd_ref[...] = jnp.sqrt(evar)
    emax_ref[...] = jnp.broadcast_to(jnp.max(me), (1, 1))
    emin_ref[...] = jnp.broadcast_to(jnp.min(me), (1, 1))
    ed = en - de
    eae_ref[...] = jnp.abs(ed)
    ese_ref[...] = ed * ed


def _tc_finalize(me, de2, part, interpret=False):
    b = me.shape[1]
    vb = jax.ShapeDtypeStruct((1, b), jnp.float32)
    sb = jax.ShapeDtypeStruct((1, 1), jnp.float32)
    return pl.pallas_call(
        _tc_finalize_body,
        out_shape=(vb, sb, sb, vb, vb, vb, vb, vb, vb, vb, vb),
        interpret=interpret,
    )(me, de2, part)


def kernel(model_energies, model_forces, data_energy, data_forces, image_idx):
    m, n, _ = model_forces.shape
    b = model_energies.shape[1]
    info = plsc.get_sparse_core_info()
    nc, ns = info.num_cores, info.num_subcores
    planes = [model_forces[mm, :, cc] for mm in range(m) for cc in range(3)]
    dplanes = [data_forces[:, cc] for cc in range(3)]
    fx, fy, fz, part = _sc_forces(planes, dplanes, image_idx, b, nc, ns)
    forces = jnp.stack([fx, fy, fz], axis=-1)
    (en, emax, emin, evar, esd, fvar, fsd, eae, ese, fae, fse) = _tc_finalize(
        model_energies, data_energy.reshape(1, b), part.reshape(nc * ns, 4, b))
    return (en.reshape(b), forces, emax.reshape(1),
            emin.reshape(1), evar.reshape(b), esd.reshape(b), fvar.reshape(b),
            fsd.reshape(b), eae.reshape(b), ese.reshape(b), fae.reshape(b),
            fse.reshape(b))
